# baseline (speedup 1.0000x reference)
import jax
import jax.numpy as jnp
from jax import lax
from jax.experimental import pallas as pl
from jax.experimental.pallas import tpu as pltpu

BT = 1024


def _conv1_rowgroup(w1):
    wk = w1.reshape(6, 6, 5)
    er = (jnp.arange(2)[:, None, None] + jnp.arange(6)[None, :, None]
          == jnp.arange(7)[None, None, :]).astype(jnp.float32)
    ej = (2 * jnp.arange(11)[:, None, None, None]
          + jnp.arange(2)[None, :, None, None]
          + jnp.arange(6)[None, None, :, None]
          == jnp.arange(28)[None, None, None, :]).astype(jnp.float32)
    m = jnp.einsum('abo,rai->orbi', wk, er)
    wg = jnp.einsum('orbi,pqbc->rqopic', m, ej)
    wg = wg.reshape(4, 55, 196)
    wg = jnp.pad(wg, ((0, 0), (0, 1), (0, 0)))
    return wg.reshape(224, 196)


def _conv2_rowgroup(w2):
    wk = w2.reshape(5, 5, 5, 16)
    er = (jnp.arange(2)[:, None, None] + jnp.arange(5)[None, :, None]
          == jnp.arange(6)[None, None, :]).astype(jnp.float32)
    ej = (2 * jnp.arange(3)[:, None, None, None]
          + jnp.arange(2)[None, :, None, None]
          + jnp.arange(5)[None, None, :, None]
          == jnp.arange(11)[None, None, None, :]).astype(jnp.float32)
    m = jnp.einsum('cabo,rai->ocrbi', wk, er)
    wg = jnp.einsum('ocrbi,pqbs->rqopics', m, ej)
    wg = wg.reshape(4, 48, 6, 55)
    wg = jnp.pad(wg, ((0, 0), (0, 0), (0, 0), (0, 1)))
    return wg.reshape(192, 336)


def _fused_kernel(x_ref, wc1_ref, b1_ref, wc2_ref, b2_ref,
                  wl1_ref, bl1_ref, wl2_ref, bl2_ref, wl3_ref, bl3_ref,
                  out_ref):
    wg = wc1_ref[...]
    cols = []
    for g in range(11):
        cg = jnp.dot(wg, x_ref[pl.ds(g * 56, 196), :],
                     preferred_element_type=jnp.float32)
        m01 = jnp.maximum(cg[0:56], cg[56:112])
        m23 = jnp.maximum(cg[112:168], cg[168:224])
        cols.append(jnp.maximum(m01, m23))
    t = jnp.stack(cols, axis=0)
    p1 = jnp.maximum(t + b1_ref[...].reshape(1, 56, 1), 0.0)
    r1 = p1.reshape(11 * 56, BT)

    wg2 = wc2_ref[...]
    rows = []
    for g2 in range(3):
        cg = jnp.dot(wg2, r1[g2 * 112:g2 * 112 + 336, :],
                     preferred_element_type=jnp.float32)
        m01 = jnp.maximum(cg[0:48], cg[48:96])
        m23 = jnp.maximum(cg[96:144], cg[144:192])
        rows.append(jnp.maximum(m01, m23))
    t = jnp.stack(rows, axis=0)
    p2 = jnp.maximum(t + b2_ref[...].reshape(1, 48, 1), 0.0)
    f = p2.reshape(144, BT)

    h1 = jnp.maximum(
        jnp.dot(wl1_ref[...], f, preferred_element_type=jnp.float32)
        + bl1_ref[...], 0.0)
    h2 = jnp.maximum(
        jnp.dot(wl2_ref[...], h1, preferred_element_type=jnp.float32)
        + bl2_ref[...], 0.0)
    out_ref[...] = (jnp.dot(wl3_ref[...], h2, preferred_element_type=jnp.float32)
                    + bl3_ref[...])


def kernel(w1, b1, w2, b2, wl1, bl1, wl2, bl2, wl3, bl3, x):
    n = x.shape[0]
    npad = ((n + BT - 1) // BT) * BT
    xt = x.reshape(n, 28 * 28).T
    if npad != n:
        xt = jnp.pad(xt, ((0, 0), (0, npad - n)))

    wc1 = _conv1_rowgroup(w1)
    wc2 = _conv2_rowgroup(w2)
    b1e = jnp.pad(jnp.repeat(b1.reshape(5), 11), (0, 1)).reshape(56, 1)
    b2e = jnp.repeat(b2.reshape(16), 3).reshape(48, 1)
    wfc1 = jnp.transpose(wl1.reshape(3, 3, 512, 16),
                         (2, 0, 3, 1)).reshape(512, 144)

    def resident(shape):
        nd = len(shape)
        return pl.BlockSpec(shape, lambda b, _nd=nd: (0,) * _nd)

    in_specs = [
        pl.BlockSpec((28 * 28, BT), lambda b: (0, b)),
        resident((224, 196)), resident((56, 1)),
        resident((192, 336)), resident((48, 1)),
        resident((512, 144)), resident((512, 1)),
        resident((512, 512)), resident((512, 1)),
        resident((10, 512)), resident((10, 1)),
    ]

    out = pl.pallas_call(
        _fused_kernel,
        grid=(npad // BT,),
        in_specs=in_specs,
        out_specs=pl.BlockSpec((10, BT), lambda b: (0, b)),
        out_shape=jax.ShapeDtypeStruct((10, npad), jnp.float32),
        compiler_params=pltpu.CompilerParams(
            dimension_semantics=("parallel",),
            vmem_limit_bytes=64 * 1024 * 1024),
    )(xt, wc1, b1e, wc2, b2e, wfc1, bl1, wl2, bl2, wl3, bl3)
    return out.T[:n]


# speedup vs baseline: 1.0097x; 1.0097x over previous
import jax
import jax.numpy as jnp
from jax import lax
from jax.experimental import pallas as pl
from jax.experimental.pallas import tpu as pltpu

BT = 2048


def _conv1_rowgroup(w1):
    wk = w1.reshape(6, 6, 5)
    er = (jnp.arange(2)[:, None, None] + jnp.arange(6)[None, :, None]
          == jnp.arange(7)[None, None, :]).astype(jnp.float32)
    ej = (2 * jnp.arange(11)[:, None, None, None]
          + jnp.arange(2)[None, :, None, None]
          + jnp.arange(6)[None, None, :, None]
          == jnp.arange(28)[None, None, None, :]).astype(jnp.float32)
    m = jnp.einsum('abo,rai->orbi', wk, er)
    wg = jnp.einsum('orbi,pqbc->rqopic', m, ej)
    wg = wg.reshape(4, 55, 196)
    wg = jnp.pad(wg, ((0, 0), (0, 1), (0, 0)))
    return wg.reshape(224, 196)


def _conv2_rowgroup(w2):
    wk = w2.reshape(5, 5, 5, 16)
    er = (jnp.arange(2)[:, None, None] + jnp.arange(5)[None, :, None]
          == jnp.arange(6)[None, None, :]).astype(jnp.float32)
    ej = (2 * jnp.arange(3)[:, None, None, None]
          + jnp.arange(2)[None, :, None, None]
          + jnp.arange(5)[None, None, :, None]
          == jnp.arange(11)[None, None, None, :]).astype(jnp.float32)
    m = jnp.einsum('cabo,rai->ocrbi', wk, er)
    wg = jnp.einsum('ocrbi,pqbs->rqopics', m, ej)
    wg = wg.reshape(4, 48, 6, 55)
    wg = jnp.pad(wg, ((0, 0), (0, 0), (0, 0), (0, 1)))
    return wg.reshape(192, 336)


def _fused_kernel(x_ref, wc1_ref, b1_ref, wc2_ref, b2_ref,
                  wl1_ref, bl1_ref, wl2_ref, bl2_ref, wl3_ref, bl3_ref,
                  out_ref):
    wg = wc1_ref[...]
    cols = []
    for g in range(11):
        cg = jnp.dot(wg, x_ref[pl.ds(g * 56, 196), :],
                     preferred_element_type=jnp.float32)
        m01 = jnp.maximum(cg[0:56], cg[56:112])
        m23 = jnp.maximum(cg[112:168], cg[168:224])
        cols.append(jnp.maximum(m01, m23))
    t = jnp.stack(cols, axis=0)
    p1 = jnp.maximum(t + b1_ref[...].reshape(1, 56, 1), 0.0)
    r1 = p1.reshape(11 * 56, BT)

    wg2 = wc2_ref[...]
    rows = []
    for g2 in range(3):
        cg = jnp.dot(wg2, r1[g2 * 112:g2 * 112 + 336, :],
                     preferred_element_type=jnp.float32)
        m01 = jnp.maximum(cg[0:48], cg[48:96])
        m23 = jnp.maximum(cg[96:144], cg[144:192])
        rows.append(jnp.maximum(m01, m23))
    t = jnp.stack(rows, axis=0)
    p2 = jnp.maximum(t + b2_ref[...].reshape(1, 48, 1), 0.0)
    f = p2.reshape(144, BT)

    h1 = jnp.maximum(
        jnp.dot(wl1_ref[...], f, preferred_element_type=jnp.float32)
        + bl1_ref[...], 0.0)
    h2 = jnp.maximum(
        jnp.dot(wl2_ref[...], h1, preferred_element_type=jnp.float32)
        + bl2_ref[...], 0.0)
    out_ref[...] = (jnp.dot(wl3_ref[...], h2, preferred_element_type=jnp.float32)
                    + bl3_ref[...])


def kernel(w1, b1, w2, b2, wl1, bl1, wl2, bl2, wl3, bl3, x):
    n = x.shape[0]
    npad = ((n + BT - 1) // BT) * BT
    xt = x.reshape(n, 28 * 28).T
    if npad != n:
        xt = jnp.pad(xt, ((0, 0), (0, npad - n)))

    wc1 = _conv1_rowgroup(w1)
    wc2 = _conv2_rowgroup(w2)
    b1e = jnp.pad(jnp.repeat(b1.reshape(5), 11), (0, 1)).reshape(56, 1)
    b2e = jnp.repeat(b2.reshape(16), 3).reshape(48, 1)
    wfc1 = jnp.transpose(wl1.reshape(3, 3, 512, 16),
                         (2, 0, 3, 1)).reshape(512, 144)

    def resident(shape):
        nd = len(shape)
        return pl.BlockSpec(shape, lambda b, _nd=nd: (0,) * _nd)

    in_specs = [
        pl.BlockSpec((28 * 28, BT), lambda b: (0, b)),
        resident((224, 196)), resident((56, 1)),
        resident((192, 336)), resident((48, 1)),
        resident((512, 144)), resident((512, 1)),
        resident((512, 512)), resident((512, 1)),
        resident((10, 512)), resident((10, 1)),
    ]

    out = pl.pallas_call(
        _fused_kernel,
        grid=(npad // BT,),
        in_specs=in_specs,
        out_specs=pl.BlockSpec((10, BT), lambda b: (0, b)),
        out_shape=jax.ShapeDtypeStruct((10, npad), jnp.float32),
        compiler_params=pltpu.CompilerParams(
            dimension_semantics=("parallel",),
            vmem_limit_bytes=64 * 1024 * 1024),
    )(xt, wc1, b1e, wc2, b2e, wfc1, bl1, wl2, bl2, wl3, bl3)
    return out.T[:n]
